# baseline (speedup 1.0000x reference)
import functools

import jax
import jax.numpy as jnp
import numpy as np
from jax import lax
from jax.experimental import pallas as pl
from jax.experimental.pallas import tpu as pltpu
from jax.experimental.pallas import tpu_sc as plsc

_N = 10000
_E = 320000
_D_IN = 128
_D_HID = 1024
_D_OUT = 128

_NCORES = 2
_NSUB = 16
_NTILES = _NCORES * _NSUB
_CHUNK = 128
_TAIL = 16

_ROWS_BLK = 512


_N_PAD = 10240
_D = 128


_NBUF = 2


def _make_seg_sum(with_counts):
    rows_per_tile = _N_PAD // _NSUB
    edges_per_tile = _E // _NTILES
    n_main = edges_per_tile // _CHUNK
    mesh = plsc.VectorSubcoreMesh(core_axis_name="c", subcore_axis_name="s")

    out_type = [jax.ShapeDtypeStruct((_NCORES, _N_PAD, _D), jnp.float32)]
    scratch = (
        [pltpu.VMEM((_CHUNK,), jnp.int32)] * 3
        + [pltpu.VMEM((_CHUNK,), jnp.int32)] * 2
        + [pltpu.VMEM((_CHUNK, _D), jnp.float32)] * 2
        + [pltpu.VMEM_SHARED((_N_PAD, _D), jnp.float32)]
        + [pltpu.SemaphoreType.DMA] * 9
        + [pltpu.VMEM((_TAIL,), jnp.int32)] * 2)
    if with_counts:
        out_type.append(jax.ShapeDtypeStruct((_NTILES, _N_PAD), jnp.float32))
        scratch.append(pltpu.VMEM((_N_PAD,), jnp.float32))

    def body(tbl, ei_h, zer, zer1, out, out_cnt,
             src_v, dst_v, rows_v, acc, sems, src_t, dst_t, hist_v):
        cid = lax.axis_index("c")
        sid = lax.axis_index("s")
        wid = sid * _NCORES + cid
        my_rows = pl.ds(sid * rows_per_tile, rows_per_tile)
        sem_g = sems[0:2]
        sem_s = sems[2:4]
        sem_si = sems[4:7]
        sem_di = sems[7:9]
        e0 = wid * edges_per_tile
        ones16 = jnp.ones((16,), jnp.float32)

        def _idx(off, ring, sem, i, b):
            return (ei_h.at[pl.ds(off + e0 + i * _CHUNK, _CHUNK)], ring[b],
                    sem[b])

        def si_issue(i, s):
            pltpu.async_copy(*_idx(0, src_v, sem_si, i, s))

        def si_wait(i, s):
            pltpu.make_async_copy(*_idx(0, src_v, sem_si, i, s)).wait()

        def di_issue(i, b):
            pltpu.async_copy(*_idx(_E, dst_v, sem_di, i, b))

        def di_wait(i, b):
            pltpu.make_async_copy(*_idx(_E, dst_v, sem_di, i, b)).wait()

        def gather(i, b, s):
            pltpu.async_copy(tbl.at[src_v[s]], rows_v[b], sem_g[b])

        def gather_wait(i, b, s):
            pltpu.make_async_copy(tbl.at[src_v[s]], rows_v[b],
                                  sem_g[b]).wait()

        def scat(i, b):
            pltpu.async_copy(rows_v[b], acc.at[dst_v[b]], sem_s[b], add=True)

        def scat_wait(i, b):
            pltpu.make_async_copy(rows_v[b], acc.at[dst_v[b]],
                                  sem_s[b]).wait()

        def hist(idx_ref, nvec):
            if with_counts:
                for k in range(nvec):
                    plsc.addupdate_scatter(hist_v, [idx_ref[pl.ds(k * 16, 16)]],
                                           ones16)

        n_grp = n_main // 6

        def stage(i, u, g):
            b, s = u % 2, u % 3
            bn, sn = (u + 1) % 2, (u + 1) % 3
            if u == 0:
                @pl.when(g > 0)
                def _():
                    scat_wait(i - 1, bn)
            else:
                scat_wait(i - 1, bn)
            if u < 5:
                di_issue(i + 1, bn)
                si_wait(i + 1, sn)
                gather(i + 1, bn, sn)
            else:
                @pl.when(g < n_grp - 1)
                def _():
                    di_issue(i + 1, bn)
                    si_wait(i + 1, sn)
                    gather(i + 1, bn, sn)
            gather_wait(i, b, s)
            if u < 3:
                si_issue(i + 3, s)
            else:
                @pl.when(g < n_grp - 1)
                def _():
                    si_issue(i + 3, s)
            di_wait(i, b)
            scat(i, b)
            hist(dst_v[b], _CHUNK // 16)

        si_issue(0, 0)
        di_issue(0, 0)
        si_issue(1, 1)
        si_issue(2, 2)
        pltpu.sync_copy(zer.at[my_rows], acc.at[my_rows])
        if with_counts:
            pltpu.sync_copy(zer1, hist_v)
        si_wait(0, 0)
        gather(0, 0, 0)
        plsc.subcore_barrier()

        def group(g, carry):
            for u in range(6):
                stage(6 * g + u, u, g)
            return carry

        lax.fori_loop(0, n_grp, group, 0)
        te = e0 + n_main * _CHUNK
        pltpu.sync_copy(ei_h.at[pl.ds(te, _TAIL)], src_t)
        pltpu.sync_copy(ei_h.at[pl.ds(_E + te, _TAIL)], dst_t)
        rows_tail = rows_v[0].at[pl.ds(0, _TAIL)]
        pltpu.async_copy(tbl.at[src_t], rows_tail, sem_g[0]).wait()
        scat_wait(n_main - 1, 1)
        pltpu.async_copy(rows_tail, acc.at[dst_t], sem_s[0], add=True).wait()
        hist(dst_t, _TAIL // 16)
        plsc.subcore_barrier()
        pltpu.sync_copy(acc.at[my_rows], out.at[cid, my_rows])
        if with_counts:
            pltpu.sync_copy(hist_v, out_cnt.at[wid])

    def _split(scr):
        return (list(scr[0:3]), list(scr[3:5]), list(scr[5:7]),
                scr[7], list(scr[8:17]), scr[17], scr[18])

    if with_counts:
        def body_c(tbl, ei_h, zer, zer1, out, out_cnt, *scr):
            sv, dv, rv, acc, sems, st, dt = _split(scr)
            body(tbl, ei_h, zer, zer1, out, out_cnt,
                 sv, dv, rv, acc, sems, st, dt, scr[-1])
        fn = body_c
    else:
        def body_n(tbl, ei_h, zer, out, *scr):
            sv, dv, rv, acc, sems, st, dt = _split(scr)
            body(tbl, ei_h, zer, None, out, None,
                 sv, dv, rv, acc, sems, st, dt, None)
        fn = body_n

    return pl.kernel(
        fn, out_type=out_type, mesh=mesh, scratch_types=scratch,
        compiler_params=pltpu.CompilerParams(needs_layout_passes=False))


_seg_sum_cache = {}


def _seg_sum(with_counts):
    if with_counts not in _seg_sum_cache:
        _seg_sum_cache[with_counts] = _make_seg_sum(with_counts)
    return _seg_sum_cache[with_counts]


def _dot(a, b):
    return jax.lax.dot_general(
        a.astype(jnp.bfloat16), b.astype(jnp.bfloat16),
        (((1,), (0,)), ((), ())), preferred_element_type=jnp.float32)


def _inv_cnt(c_r):
    ones = jnp.ones((_NTILES, 1), jnp.float32)
    cnt = jax.lax.dot_general(c_r[...], ones, (((0,), (0,)), ((), ())),
                              preferred_element_type=jnp.float32)
    return 1.0 / jnp.maximum(cnt, 1.0)


def _l1_body(x_r, ag_r, c_r, w1_r, b1_r, w2_r, p_r, q_r):
    agg = (ag_r[0] + ag_r[1]) * _inv_cnt(c_r)
    ax = jnp.concatenate([agg.astype(jnp.bfloat16),
                          x_r[...].astype(jnp.bfloat16)], axis=1)
    h = jax.lax.dot_general(ax, w1_r[...].astype(jnp.bfloat16),
                            (((1,), (0,)), ((), ())),
                            preferred_element_type=jnp.float32) + b1_r[...]
    h = jnp.maximum(h, 0.0)
    pq = _dot(h, w2_r[...])
    p_r[...] = pq[:, :_D_OUT]
    q_r[...] = pq[:, _D_OUT:]


def _layer1_fused(x, agg1, cnt, w1l, b1, w1r, w2l, w2r):
    nb = _N_PAD // _ROWS_BLK
    row_spec = lambda w: pl.BlockSpec((_ROWS_BLK, w), lambda i: (i, 0))
    full_spec = lambda r, c: pl.BlockSpec((r, c), lambda i: (0, 0))
    w1 = jnp.concatenate([w1l, w1r], axis=0)
    w2 = jnp.concatenate([w2l, w2r], axis=1)
    return pl.pallas_call(
        _l1_body,
        grid=(nb,),
        in_specs=[
            row_spec(_D_IN),
            pl.BlockSpec((2, _ROWS_BLK, _D), lambda i: (0, i, 0)),
            pl.BlockSpec((_NTILES, _ROWS_BLK), lambda i: (0, i)),
            full_spec(2 * _D_IN, _D_HID), full_spec(1, _D_HID),
            full_spec(_D_HID, 2 * _D_OUT),
        ],
        out_specs=[row_spec(_D_OUT), row_spec(_D_OUT)],
        out_shape=[
            jax.ShapeDtypeStruct((_N, _D_OUT), jnp.float32),
            jax.ShapeDtypeStruct((_N, _D_OUT), jnp.float32),
        ],
    )(x, agg1, cnt, w1, b1.reshape(1, _D_HID), w2)


def _l2_body(ag_r, c_r, q_r, b2_r, out_r):
    o = (ag_r[0] + ag_r[1]) * _inv_cnt(c_r) + b2_r[...] + q_r[...]
    m = jnp.max(o, axis=1, keepdims=True)
    s = jnp.sum(jnp.exp(o - m), axis=1, keepdims=True)
    out_r[...] = o - m - jnp.log(s)


def _layer2_final(agg2, cnt, q, b2):
    nb = _N_PAD // _ROWS_BLK
    row_spec = lambda w: pl.BlockSpec((_ROWS_BLK, w), lambda i: (i, 0))
    return pl.pallas_call(
        _l2_body,
        grid=(nb,),
        in_specs=[
            pl.BlockSpec((2, _ROWS_BLK, _D), lambda i: (0, i, 0)),
            pl.BlockSpec((_NTILES, _ROWS_BLK), lambda i: (0, i)),
            row_spec(_D_OUT),
            pl.BlockSpec((1, _D_OUT), lambda i: (0, 0)),
        ],
        out_specs=row_spec(_D_OUT),
        out_shape=jax.ShapeDtypeStruct((_N, _D_OUT), jnp.float32),
    )(agg2, cnt, q, b2.reshape(1, _D_OUT))


def kernel(x, edge_index, W1_l, b1, W1_r, W2_l, b2, W2_r):
    ei = edge_index.reshape(-1)

    zer = np.zeros((_N_PAD, _D), np.float32)
    zer1 = np.zeros((_N_PAD,), np.float32)
    agg1, cnt = _seg_sum(True)(x, ei, zer, zer1)

    p, q = _layer1_fused(x, agg1, cnt, W1_l, b1, W1_r, W2_l, W2_r)

    (agg2,) = _seg_sum(False)(p, ei, zer)

    return _layer2_final(agg2, cnt, q, b2)


# speedup vs baseline: 1.0626x; 1.0626x over previous
import functools

import jax
import jax.numpy as jnp
import numpy as np
from jax import lax
from jax.experimental import pallas as pl
from jax.experimental.pallas import tpu as pltpu
from jax.experimental.pallas import tpu_sc as plsc

_N = 10000
_E = 320000
_D_IN = 128
_D_HID = 1024
_D_OUT = 128

_NCORES = 2
_NSUB = 16
_NTILES = _NCORES * _NSUB
_CHUNK = 128
_TAIL = 16

_ROWS_BLK = 2048


_N_PAD = 10240
_D = 128


_NBUF = 2


def _make_seg_sum(with_counts):
    rows_per_tile = _N_PAD // _NSUB
    edges_per_tile = _E // _NTILES
    n_main = edges_per_tile // _CHUNK
    mesh = plsc.VectorSubcoreMesh(core_axis_name="c", subcore_axis_name="s")

    out_type = [jax.ShapeDtypeStruct((_NCORES, _N_PAD, _D), jnp.float32)]
    scratch = (
        [pltpu.VMEM((_CHUNK,), jnp.int32)] * 3
        + [pltpu.VMEM((_CHUNK,), jnp.int32)] * 2
        + [pltpu.VMEM((_CHUNK, _D), jnp.float32)] * 2
        + [pltpu.VMEM_SHARED((_N_PAD, _D), jnp.float32)]
        + [pltpu.SemaphoreType.DMA] * 9
        + [pltpu.VMEM((_TAIL,), jnp.int32)] * 2)
    if with_counts:
        out_type.append(jax.ShapeDtypeStruct((_NTILES, _N_PAD), jnp.float32))
        scratch.append(pltpu.VMEM((_N_PAD,), jnp.float32))

    def body(tbl, ei_h, zer, zer1, out, out_cnt,
             src_v, dst_v, rows_v, acc, sems, src_t, dst_t, hist_v):
        cid = lax.axis_index("c")
        sid = lax.axis_index("s")
        wid = sid * _NCORES + cid
        my_rows = pl.ds(sid * rows_per_tile, rows_per_tile)
        sem_g = sems[0:2]
        sem_s = sems[2:4]
        sem_si = sems[4:7]
        sem_di = sems[7:9]
        e0 = wid * edges_per_tile
        ones16 = jnp.ones((16,), jnp.float32)

        def _idx(off, ring, sem, i, b):
            return (ei_h.at[pl.ds(off + e0 + i * _CHUNK, _CHUNK)], ring[b],
                    sem[b])

        def si_issue(i, s):
            pltpu.async_copy(*_idx(0, src_v, sem_si, i, s))

        def si_wait(i, s):
            pltpu.make_async_copy(*_idx(0, src_v, sem_si, i, s)).wait()

        def di_issue(i, b):
            pltpu.async_copy(*_idx(_E, dst_v, sem_di, i, b))

        def di_wait(i, b):
            pltpu.make_async_copy(*_idx(_E, dst_v, sem_di, i, b)).wait()

        def gather(i, b, s):
            pltpu.async_copy(tbl.at[src_v[s]], rows_v[b], sem_g[b])

        def gather_wait(i, b, s):
            pltpu.make_async_copy(tbl.at[src_v[s]], rows_v[b],
                                  sem_g[b]).wait()

        def scat(i, b):
            pltpu.async_copy(rows_v[b], acc.at[dst_v[b]], sem_s[b], add=True)

        def scat_wait(i, b):
            pltpu.make_async_copy(rows_v[b], acc.at[dst_v[b]],
                                  sem_s[b]).wait()

        def hist(idx_ref, nvec):
            if with_counts:
                for k in range(nvec):
                    plsc.addupdate_scatter(hist_v, [idx_ref[pl.ds(k * 16, 16)]],
                                           ones16)

        n_grp = n_main // 6

        def stage(i, u, g):
            b, s = u % 2, u % 3
            bn, sn = (u + 1) % 2, (u + 1) % 3
            if u == 0:
                @pl.when(g > 0)
                def _():
                    scat_wait(i - 1, bn)
            else:
                scat_wait(i - 1, bn)
            if u < 5:
                di_issue(i + 1, bn)
                si_wait(i + 1, sn)
                gather(i + 1, bn, sn)
            else:
                @pl.when(g < n_grp - 1)
                def _():
                    di_issue(i + 1, bn)
                    si_wait(i + 1, sn)
                    gather(i + 1, bn, sn)
            gather_wait(i, b, s)
            if u < 3:
                si_issue(i + 3, s)
            else:
                @pl.when(g < n_grp - 1)
                def _():
                    si_issue(i + 3, s)
            di_wait(i, b)
            scat(i, b)
            hist(dst_v[b], _CHUNK // 16)

        si_issue(0, 0)
        di_issue(0, 0)
        si_issue(1, 1)
        si_issue(2, 2)
        pltpu.sync_copy(zer.at[my_rows], acc.at[my_rows])
        if with_counts:
            pltpu.sync_copy(zer1, hist_v)
        si_wait(0, 0)
        gather(0, 0, 0)
        plsc.subcore_barrier()

        def group(g, carry):
            for u in range(6):
                stage(6 * g + u, u, g)
            return carry

        lax.fori_loop(0, n_grp, group, 0)
        te = e0 + n_main * _CHUNK
        pltpu.sync_copy(ei_h.at[pl.ds(te, _TAIL)], src_t)
        pltpu.sync_copy(ei_h.at[pl.ds(_E + te, _TAIL)], dst_t)
        rows_tail = rows_v[0].at[pl.ds(0, _TAIL)]
        pltpu.async_copy(tbl.at[src_t], rows_tail, sem_g[0]).wait()
        scat_wait(n_main - 1, 1)
        pltpu.async_copy(rows_tail, acc.at[dst_t], sem_s[0], add=True).wait()
        hist(dst_t, _TAIL // 16)
        plsc.subcore_barrier()
        pltpu.sync_copy(acc.at[my_rows], out.at[cid, my_rows])
        if with_counts:
            pltpu.sync_copy(hist_v, out_cnt.at[wid])

    def _split(scr):
        return (list(scr[0:3]), list(scr[3:5]), list(scr[5:7]),
                scr[7], list(scr[8:17]), scr[17], scr[18])

    if with_counts:
        def body_c(tbl, ei_h, zer, zer1, out, out_cnt, *scr):
            sv, dv, rv, acc, sems, st, dt = _split(scr)
            body(tbl, ei_h, zer, zer1, out, out_cnt,
                 sv, dv, rv, acc, sems, st, dt, scr[-1])
        fn = body_c
    else:
        def body_n(tbl, ei_h, zer, out, *scr):
            sv, dv, rv, acc, sems, st, dt = _split(scr)
            body(tbl, ei_h, zer, None, out, None,
                 sv, dv, rv, acc, sems, st, dt, None)
        fn = body_n

    return pl.kernel(
        fn, out_type=out_type, mesh=mesh, scratch_types=scratch,
        compiler_params=pltpu.CompilerParams(needs_layout_passes=False))


_seg_sum_cache = {}


def _seg_sum(with_counts):
    if with_counts not in _seg_sum_cache:
        _seg_sum_cache[with_counts] = _make_seg_sum(with_counts)
    return _seg_sum_cache[with_counts]


def _dot(a, b):
    return jax.lax.dot_general(
        a.astype(jnp.bfloat16), b.astype(jnp.bfloat16),
        (((1,), (0,)), ((), ())), preferred_element_type=jnp.float32)


def _inv_cnt(c_r):
    ones = jnp.ones((_NTILES, 1), jnp.float32)
    cnt = jax.lax.dot_general(c_r[...], ones, (((0,), (0,)), ((), ())),
                              preferred_element_type=jnp.float32)
    return 1.0 / jnp.maximum(cnt, 1.0)


def _l1_body(x_r, ag_r, c_r, w1_r, b1_r, w2_r, p_r, q_r):
    agg = (ag_r[0] + ag_r[1]) * _inv_cnt(c_r)
    ax = jnp.concatenate([agg.astype(jnp.bfloat16),
                          x_r[...].astype(jnp.bfloat16)], axis=1)
    h = jax.lax.dot_general(ax, w1_r[...].astype(jnp.bfloat16),
                            (((1,), (0,)), ((), ())),
                            preferred_element_type=jnp.float32) + b1_r[...]
    h = jnp.maximum(h, 0.0)
    pq = _dot(h, w2_r[...])
    p_r[...] = pq[:, :_D_OUT]
    q_r[...] = pq[:, _D_OUT:]


def _layer1_fused(x, agg1, cnt, w1l, b1, w1r, w2l, w2r):
    nb = _N_PAD // _ROWS_BLK
    row_spec = lambda w: pl.BlockSpec((_ROWS_BLK, w), lambda i: (i, 0))
    full_spec = lambda r, c: pl.BlockSpec((r, c), lambda i: (0, 0))
    w1 = jnp.concatenate([w1l, w1r], axis=0)
    w2 = jnp.concatenate([w2l, w2r], axis=1)
    return pl.pallas_call(
        _l1_body,
        grid=(nb,),
        in_specs=[
            row_spec(_D_IN),
            pl.BlockSpec((2, _ROWS_BLK, _D), lambda i: (0, i, 0)),
            pl.BlockSpec((_NTILES, _ROWS_BLK), lambda i: (0, i)),
            full_spec(2 * _D_IN, _D_HID), full_spec(1, _D_HID),
            full_spec(_D_HID, 2 * _D_OUT),
        ],
        out_specs=[row_spec(_D_OUT), row_spec(_D_OUT)],
        out_shape=[
            jax.ShapeDtypeStruct((_N, _D_OUT), jnp.float32),
            jax.ShapeDtypeStruct((_N, _D_OUT), jnp.float32),
        ],
    )(x, agg1, cnt, w1, b1.reshape(1, _D_HID), w2)


def _l2_body(ag_r, c_r, q_r, b2_r, out_r):
    o = (ag_r[0] + ag_r[1]) * _inv_cnt(c_r) + b2_r[...] + q_r[...]
    m = jnp.max(o, axis=1, keepdims=True)
    s = jnp.sum(jnp.exp(o - m), axis=1, keepdims=True)
    out_r[...] = o - m - jnp.log(s)


def _layer2_final(agg2, cnt, q, b2):
    nb = _N_PAD // _ROWS_BLK
    row_spec = lambda w: pl.BlockSpec((_ROWS_BLK, w), lambda i: (i, 0))
    return pl.pallas_call(
        _l2_body,
        grid=(nb,),
        in_specs=[
            pl.BlockSpec((2, _ROWS_BLK, _D), lambda i: (0, i, 0)),
            pl.BlockSpec((_NTILES, _ROWS_BLK), lambda i: (0, i)),
            row_spec(_D_OUT),
            pl.BlockSpec((1, _D_OUT), lambda i: (0, 0)),
        ],
        out_specs=row_spec(_D_OUT),
        out_shape=jax.ShapeDtypeStruct((_N, _D_OUT), jnp.float32),
    )(agg2, cnt, q, b2.reshape(1, _D_OUT))


def kernel(x, edge_index, W1_l, b1, W1_r, W2_l, b2, W2_r):
    ei = edge_index.reshape(-1)

    zer = np.zeros((_N_PAD, _D), np.float32)
    zer1 = np.zeros((_N_PAD,), np.float32)
    agg1, cnt = _seg_sum(True)(x, ei, zer, zer1)

    p, q = _layer1_fused(x, agg1, cnt, W1_l, b1, W1_r, W2_l, W2_r)

    (agg2,) = _seg_sum(False)(p, ei, zer)

    return _layer2_final(agg2, cnt, q, b2)


# speedup vs baseline: 1.0645x; 1.0018x over previous
import functools

import jax
import jax.numpy as jnp
import numpy as np
from jax import lax
from jax.experimental import pallas as pl
from jax.experimental.pallas import tpu as pltpu
from jax.experimental.pallas import tpu_sc as plsc

_N = 10000
_E = 320000
_D_IN = 128
_D_HID = 1024
_D_OUT = 128

_NCORES = 2
_NSUB = 16
_NTILES = _NCORES * _NSUB
_CHUNK = 128
_TAIL = 16

_ROWS_BLK = 2560


_N_PAD = 10240
_D = 128


_NBUF = 2


def _make_seg_sum(with_counts):
    rows_per_tile = _N_PAD // _NSUB
    edges_per_tile = _E // _NTILES
    n_main = edges_per_tile // _CHUNK
    mesh = plsc.VectorSubcoreMesh(core_axis_name="c", subcore_axis_name="s")

    out_type = [jax.ShapeDtypeStruct((_NCORES, _N_PAD, _D), jnp.float32)]
    scratch = (
        [pltpu.VMEM((_CHUNK,), jnp.int32)] * 3
        + [pltpu.VMEM((_CHUNK,), jnp.int32)] * 2
        + [pltpu.VMEM((_CHUNK, _D), jnp.float32)] * 2
        + [pltpu.VMEM_SHARED((_N_PAD, _D), jnp.float32)]
        + [pltpu.SemaphoreType.DMA] * 9
        + [pltpu.VMEM((_TAIL,), jnp.int32)] * 2)
    if with_counts:
        out_type.append(jax.ShapeDtypeStruct((_NTILES, _N_PAD), jnp.float32))
        scratch.append(pltpu.VMEM((_N_PAD,), jnp.float32))

    def body(tbl, ei_h, zer, zer1, out, out_cnt,
             src_v, dst_v, rows_v, acc, sems, src_t, dst_t, hist_v):
        cid = lax.axis_index("c")
        sid = lax.axis_index("s")
        wid = sid * _NCORES + cid
        my_rows = pl.ds(sid * rows_per_tile, rows_per_tile)
        sem_g = sems[0:2]
        sem_s = sems[2:4]
        sem_si = sems[4:7]
        sem_di = sems[7:9]
        e0 = wid * edges_per_tile
        ones16 = jnp.ones((16,), jnp.float32)

        def _idx(off, ring, sem, i, b):
            return (ei_h.at[pl.ds(off + e0 + i * _CHUNK, _CHUNK)], ring[b],
                    sem[b])

        def si_issue(i, s):
            pltpu.async_copy(*_idx(0, src_v, sem_si, i, s))

        def si_wait(i, s):
            pltpu.make_async_copy(*_idx(0, src_v, sem_si, i, s)).wait()

        def di_issue(i, b):
            pltpu.async_copy(*_idx(_E, dst_v, sem_di, i, b))

        def di_wait(i, b):
            pltpu.make_async_copy(*_idx(_E, dst_v, sem_di, i, b)).wait()

        def gather(i, b, s):
            pltpu.async_copy(tbl.at[src_v[s]], rows_v[b], sem_g[b])

        def gather_wait(i, b, s):
            pltpu.make_async_copy(tbl.at[src_v[s]], rows_v[b],
                                  sem_g[b]).wait()

        def scat(i, b):
            pltpu.async_copy(rows_v[b], acc.at[dst_v[b]], sem_s[b], add=True)

        def scat_wait(i, b):
            pltpu.make_async_copy(rows_v[b], acc.at[dst_v[b]],
                                  sem_s[b]).wait()

        def hist(idx_ref, nvec):
            if with_counts:
                for k in range(nvec):
                    plsc.addupdate_scatter(hist_v, [idx_ref[pl.ds(k * 16, 16)]],
                                           ones16)

        n_grp = n_main // 6

        def stage(i, u, g):
            b, s = u % 2, u % 3
            bn, sn = (u + 1) % 2, (u + 1) % 3
            if u == 0:
                @pl.when(g > 0)
                def _():
                    scat_wait(i - 1, bn)
            else:
                scat_wait(i - 1, bn)
            if u < 5:
                di_issue(i + 1, bn)
                si_wait(i + 1, sn)
                gather(i + 1, bn, sn)
            else:
                @pl.when(g < n_grp - 1)
                def _():
                    di_issue(i + 1, bn)
                    si_wait(i + 1, sn)
                    gather(i + 1, bn, sn)
            gather_wait(i, b, s)
            if u < 3:
                si_issue(i + 3, s)
            else:
                @pl.when(g < n_grp - 1)
                def _():
                    si_issue(i + 3, s)
            di_wait(i, b)
            scat(i, b)
            hist(dst_v[b], _CHUNK // 16)

        si_issue(0, 0)
        di_issue(0, 0)
        si_issue(1, 1)
        si_issue(2, 2)
        pltpu.sync_copy(zer.at[my_rows], acc.at[my_rows])
        if with_counts:
            pltpu.sync_copy(zer1, hist_v)
        si_wait(0, 0)
        gather(0, 0, 0)
        plsc.subcore_barrier()

        def group(g, carry):
            for u in range(6):
                stage(6 * g + u, u, g)
            return carry

        lax.fori_loop(0, n_grp, group, 0)
        te = e0 + n_main * _CHUNK
        pltpu.sync_copy(ei_h.at[pl.ds(te, _TAIL)], src_t)
        pltpu.sync_copy(ei_h.at[pl.ds(_E + te, _TAIL)], dst_t)
        rows_tail = rows_v[0].at[pl.ds(0, _TAIL)]
        pltpu.async_copy(tbl.at[src_t], rows_tail, sem_g[0]).wait()
        scat_wait(n_main - 1, 1)
        pltpu.async_copy(rows_tail, acc.at[dst_t], sem_s[0], add=True).wait()
        hist(dst_t, _TAIL // 16)
        plsc.subcore_barrier()
        pltpu.sync_copy(acc.at[my_rows], out.at[cid, my_rows])
        if with_counts:
            pltpu.sync_copy(hist_v, out_cnt.at[wid])

    def _split(scr):
        return (list(scr[0:3]), list(scr[3:5]), list(scr[5:7]),
                scr[7], list(scr[8:17]), scr[17], scr[18])

    if with_counts:
        def body_c(tbl, ei_h, zer, zer1, out, out_cnt, *scr):
            sv, dv, rv, acc, sems, st, dt = _split(scr)
            body(tbl, ei_h, zer, zer1, out, out_cnt,
                 sv, dv, rv, acc, sems, st, dt, scr[-1])
        fn = body_c
    else:
        def body_n(tbl, ei_h, zer, out, *scr):
            sv, dv, rv, acc, sems, st, dt = _split(scr)
            body(tbl, ei_h, zer, None, out, None,
                 sv, dv, rv, acc, sems, st, dt, None)
        fn = body_n

    return pl.kernel(
        fn, out_type=out_type, mesh=mesh, scratch_types=scratch,
        compiler_params=pltpu.CompilerParams(needs_layout_passes=False))


_seg_sum_cache = {}


def _seg_sum(with_counts):
    if with_counts not in _seg_sum_cache:
        _seg_sum_cache[with_counts] = _make_seg_sum(with_counts)
    return _seg_sum_cache[with_counts]


def _dot(a, b):
    return jax.lax.dot_general(
        a.astype(jnp.bfloat16), b.astype(jnp.bfloat16),
        (((1,), (0,)), ((), ())), preferred_element_type=jnp.float32)


def _inv_cnt(c_r):
    ones = jnp.ones((_NTILES, 1), jnp.float32)
    cnt = jax.lax.dot_general(c_r[...], ones, (((0,), (0,)), ((), ())),
                              preferred_element_type=jnp.float32)
    return 1.0 / jnp.maximum(cnt, 1.0)


def _l1_body(x_r, ag_r, c_r, w1_r, b1_r, w2_r, p_r, q_r):
    agg = (ag_r[0] + ag_r[1]) * _inv_cnt(c_r)
    ax = jnp.concatenate([agg.astype(jnp.bfloat16),
                          x_r[...].astype(jnp.bfloat16)], axis=1)
    h = jax.lax.dot_general(ax, w1_r[...].astype(jnp.bfloat16),
                            (((1,), (0,)), ((), ())),
                            preferred_element_type=jnp.float32) + b1_r[...]
    h = jnp.maximum(h, 0.0)
    pq = _dot(h, w2_r[...])
    p_r[...] = pq[:, :_D_OUT]
    q_r[...] = pq[:, _D_OUT:]


def _layer1_fused(x, agg1, cnt, w1l, b1, w1r, w2l, w2r):
    nb = _N_PAD // _ROWS_BLK
    row_spec = lambda w: pl.BlockSpec((_ROWS_BLK, w), lambda i: (i, 0))
    full_spec = lambda r, c: pl.BlockSpec((r, c), lambda i: (0, 0))
    w1 = jnp.concatenate([w1l, w1r], axis=0)
    w2 = jnp.concatenate([w2l, w2r], axis=1)
    return pl.pallas_call(
        _l1_body,
        grid=(nb,),
        in_specs=[
            row_spec(_D_IN),
            pl.BlockSpec((2, _ROWS_BLK, _D), lambda i: (0, i, 0)),
            pl.BlockSpec((_NTILES, _ROWS_BLK), lambda i: (0, i)),
            full_spec(2 * _D_IN, _D_HID), full_spec(1, _D_HID),
            full_spec(_D_HID, 2 * _D_OUT),
        ],
        out_specs=[row_spec(_D_OUT), row_spec(_D_OUT)],
        out_shape=[
            jax.ShapeDtypeStruct((_N, _D_OUT), jnp.float32),
            jax.ShapeDtypeStruct((_N, _D_OUT), jnp.float32),
        ],
    )(x, agg1, cnt, w1, b1.reshape(1, _D_HID), w2)


def _l2_body(ag_r, c_r, q_r, b2_r, out_r):
    o = (ag_r[0] + ag_r[1]) * _inv_cnt(c_r) + b2_r[...] + q_r[...]
    m = jnp.max(o, axis=1, keepdims=True)
    s = jnp.sum(jnp.exp(o - m), axis=1, keepdims=True)
    out_r[...] = o - m - jnp.log(s)


def _layer2_final(agg2, cnt, q, b2):
    nb = _N_PAD // _ROWS_BLK
    row_spec = lambda w: pl.BlockSpec((_ROWS_BLK, w), lambda i: (i, 0))
    return pl.pallas_call(
        _l2_body,
        grid=(nb,),
        in_specs=[
            pl.BlockSpec((2, _ROWS_BLK, _D), lambda i: (0, i, 0)),
            pl.BlockSpec((_NTILES, _ROWS_BLK), lambda i: (0, i)),
            row_spec(_D_OUT),
            pl.BlockSpec((1, _D_OUT), lambda i: (0, 0)),
        ],
        out_specs=row_spec(_D_OUT),
        out_shape=jax.ShapeDtypeStruct((_N, _D_OUT), jnp.float32),
    )(agg2, cnt, q, b2.reshape(1, _D_OUT))


def kernel(x, edge_index, W1_l, b1, W1_r, W2_l, b2, W2_r):
    ei = edge_index.reshape(-1)

    zer = np.zeros((_N_PAD, _D), np.float32)
    zer1 = np.zeros((_N_PAD,), np.float32)
    agg1, cnt = _seg_sum(True)(x, ei, zer, zer1)

    p, q = _layer1_fused(x, agg1, cnt, W1_l, b1, W1_r, W2_l, W2_r)

    (agg2,) = _seg_sum(False)(p, ei, zer)

    return _layer2_final(agg2, cnt, q, b2)
